# baseline (speedup 1.0000x reference)
import functools

import jax
import jax.numpy as jnp
from jax import lax
from jax.experimental import pallas as pl
from jax.experimental.pallas import tpu as pltpu
from jax.experimental.pallas import tpu_sc as plsc

N = 10000
E = 320000
H = 128
B = 8
EPS = 1e-5
LOG2E = 1.4426950408889634

NB = 10
BN_ROWS = N // NB

NC = 2
NS = 16
NW = NC * NS
EPW = E // NW
CHUNK = 40
NCHUNK = EPW // CHUNK
SRP = 624
SREM = N - NS * SRP


def _mm_body(h_ref, w_ref, b_ref, k_ref, q_ref, v_ref, s_ref):
    y = jnp.dot(h_ref[...], w_ref[...], preferred_element_type=jnp.float32)
    y = y + b_ref[...]
    k_ref[...] = y[:, 0 * H:1 * H]
    q_ref[...] = y[:, 1 * H:2 * H]
    v_ref[...] = y[:, 2 * H:3 * H]
    s_ref[...] = y[:, 3 * H:4 * H]


def _norm_mm_body(a_ref, st_ref, gb_ref, w_ref, b_ref,
                  k_ref, q_ref, v_ref, s_ref):
    m = st_ref[0] * (1.0 / N)
    var = st_ref[1] * (1.0 / N) - m * m
    inv = lax.rsqrt(var + EPS)
    h = (a_ref[...] - m) * (inv * gb_ref[0]) + gb_ref[1]
    y = jnp.dot(h, w_ref[...], preferred_element_type=jnp.float32)
    y = y + b_ref[...]
    k_ref[...] = y[:, 0 * H:1 * H]
    q_ref[...] = y[:, 1 * H:2 * H]
    v_ref[...] = y[:, 2 * H:3 * H]
    s_ref[...] = y[:, 3 * H:4 * H]


def _post_body(p_ref, s_ref, a_ref, st_ref):
    i = pl.program_id(0)
    t = p_ref[0] + p_ref[1] + s_ref[...]
    a = jnp.where(t >= 0, t, 0.01 * t)
    a_ref[...] = a
    ssum = jnp.sum(a, axis=0, keepdims=True)
    ssq = jnp.sum(a * a, axis=0, keepdims=True)
    st = jnp.concatenate([ssum, ssq], axis=0)

    @pl.when(i == 0)
    def _():
        st_ref[...] = st

    @pl.when(i != 0)
    def _():
        st_ref[...] += st


def _final_body(a_ref, st_ref, gb_ref, batch_ref, clin_ref,
                wh_ref, wc_ref, bc_ref, o_ref):
    m = st_ref[0] * (1.0 / N)
    var = st_ref[1] * (1.0 / N) - m * m
    inv = lax.rsqrt(var + EPS)
    hn = (a_ref[...] - m) * (inv * gb_ref[0]) + gb_ref[1]
    oh = (batch_ref[...] == lax.broadcasted_iota(jnp.int32, (N, B), 1))
    oh = oh.astype(jnp.float32)
    psum = lax.dot_general(oh, hn, (((0,), (0,)), ((), ())),
                           preferred_element_type=jnp.float32)
    cnt = jnp.sum(oh, axis=0)[:, None]
    pooled = psum / jnp.clip(cnt, 1.0, None)
    out = jnp.dot(pooled, wh_ref[...], preferred_element_type=jnp.float32)
    out = out + jnp.dot(clin_ref[...], wc_ref[...],
                        preferred_element_type=jnp.float32)
    o_ref[...] = out + bc_ref[...]


def _mm_call(h, w4t, b4):
    return pl.pallas_call(
        _mm_body,
        grid=(NB,),
        in_specs=[
            pl.BlockSpec((BN_ROWS, H), lambda i: (i, 0)),
            pl.BlockSpec((H, 4 * H), lambda i: (0, 0)),
            pl.BlockSpec((1, 4 * H), lambda i: (0, 0)),
        ],
        out_specs=[pl.BlockSpec((BN_ROWS, H), lambda i: (i, 0))] * 4,
        out_shape=[jax.ShapeDtypeStruct((N, H), jnp.float32)] * 4,
    )(h, w4t, b4)


def _norm_mm_call(a, st, gb, w4t, b4):
    return pl.pallas_call(
        _norm_mm_body,
        grid=(NB,),
        in_specs=[
            pl.BlockSpec((BN_ROWS, H), lambda i: (i, 0)),
            pl.BlockSpec((2, H), lambda i: (0, 0)),
            pl.BlockSpec((2, H), lambda i: (0, 0)),
            pl.BlockSpec((H, 4 * H), lambda i: (0, 0)),
            pl.BlockSpec((1, 4 * H), lambda i: (0, 0)),
        ],
        out_specs=[pl.BlockSpec((BN_ROWS, H), lambda i: (i, 0))] * 4,
        out_shape=[jax.ShapeDtypeStruct((N, H), jnp.float32)] * 4,
    )(a, st, gb, w4t, b4)


def _post_call(parts, s):
    return pl.pallas_call(
        _post_body,
        grid=(NB,),
        in_specs=[
            pl.BlockSpec((2, BN_ROWS, H), lambda i: (0, i, 0)),
            pl.BlockSpec((BN_ROWS, H), lambda i: (i, 0)),
        ],
        out_specs=[
            pl.BlockSpec((BN_ROWS, H), lambda i: (i, 0)),
            pl.BlockSpec((2, H), lambda i: (0, 0)),
        ],
        out_shape=[
            jax.ShapeDtypeStruct((N, H), jnp.float32),
            jax.ShapeDtypeStruct((2, H), jnp.float32),
        ],
    )(parts, s)


def _final_call(a, st, gb, batch2d, clinical, wht, wct, bc):
    return pl.pallas_call(
        _final_body,
        out_shape=jax.ShapeDtypeStruct((B, 2), jnp.float32),
    )(a, st, gb, batch2d, clinical, wht, wct, bc)


def _sc_edge_body(k_hbm, q_hbm, v_hbm, src_hbm, dst_hbm, ea_hbm, we_hbm,
                  z_hbm, out_hbm,
                  sidx0, didx0, eav0, kb0, qb0, vb0,
                  sidx1, didx1, eav1, kb1, qb1, vb1,
                  wev, agg, sem0, sem1):
    cid = lax.axis_index("c")
    sid = lax.axis_index("s")
    wid = sid * NC + cid
    sets = ((sidx0, didx0, eav0, kb0, qb0, vb0, sem0),
            (sidx1, didx1, eav1, kb1, qb1, vb1, sem1))

    pltpu.sync_copy(we_hbm, wev)

    zbase = pl.multiple_of(sid * SRP, 8)
    pltpu.sync_copy(z_hbm.at[pl.ds(0, SRP)], agg.at[pl.ds(zbase, SRP)])

    @pl.when(sid == NS - 1)
    def _():
        pltpu.sync_copy(z_hbm.at[pl.ds(0, SREM)],
                        agg.at[pl.ds(NS * SRP, SREM)])

    plsc.subcore_barrier()

    we_rows = [[wev[c, pl.ds(s * 16, 16)] for s in range(8)]
               for c in range(4)]

    ebase = wid * EPW

    def load_idx(g, st):
        base = ebase + g * CHUNK
        pltpu.sync_copy(src_hbm.at[pl.ds(base, CHUNK)], st[0])
        pltpu.sync_copy(dst_hbm.at[pl.ds(base, CHUNK)], st[1])
        pltpu.sync_copy(ea_hbm.at[pl.ds(base, CHUNK)], st[2])

    def fire_gathers(st):
        pltpu.async_copy(k_hbm.at[st[1]], st[3], st[6])
        pltpu.async_copy(q_hbm.at[st[0]], st[4], st[6])
        pltpu.async_copy(v_hbm.at[st[0]], st[5], st[6])

    def wait_gathers(st):
        pltpu.make_async_copy(k_hbm.at[st[1]], st[3], st[6]).wait()
        pltpu.make_async_copy(q_hbm.at[st[0]], st[4], st[6]).wait()
        pltpu.make_async_copy(v_hbm.at[st[0]], st[5], st[6]).wait()

    def process(st):
        eav, kb, qb, vb = st[2], st[3], st[4], st[5]

        @plsc.parallel_loop(0, CHUNK, step=1, unroll=2)
        def edge_body(j):
            ear = eav[j, pl.ds(0, 16)]
            ea0 = ear[0]
            ea1 = ear[1]
            ea2 = ear[2]
            ea3 = ear[3]
            for s in range(8):
                ds = pl.ds(s * 16, 16)
                t01 = ea0 * we_rows[0][s] + ea1 * we_rows[1][s]
                t23 = ea2 * we_rows[2][s] + ea3 * we_rows[3][s]
                kq = kb[j, ds] + qb[j, ds]
                zn = (t01 + t23) + kq
                gate = 1.0 / (1.0 + jnp.exp(zn))
                vb[j, ds] = gate * vb[j, ds]

        pltpu.sync_copy(vb, agg.at[st[1]], add=True)

    load_idx(0, sets[0])
    fire_gathers(sets[0])

    def pair_body(g2, carry):
        for b in range(2):
            gi = g2 * 2 + b
            nxt = sets[1 - b]

            @pl.when(gi + 1 < NCHUNK)
            def _():
                load_idx(gi + 1, nxt)
                fire_gathers(nxt)

            wait_gathers(sets[b])
            process(sets[b])
        return carry

    lax.fori_loop(0, NCHUNK // 2, pair_body, 0)
    plsc.subcore_barrier()
    obase = pl.multiple_of(sid * SRP, 8)
    pltpu.sync_copy(agg.at[pl.ds(obase, SRP)],
                    out_hbm.at[cid, pl.ds(obase, SRP)])

    @pl.when(sid == NS - 1)
    def _():
        pltpu.sync_copy(agg.at[pl.ds(NS * SRP, SREM)],
                        out_hbm.at[cid, pl.ds(NS * SRP, SREM)])


def _sc_edge_call(k, q, v, src, dst, ea16, wetb, zrows):
    mesh = plsc.VectorSubcoreMesh(core_axis_name="c", subcore_axis_name="s")
    fn = functools.partial(
        pl.kernel,
        mesh=mesh,
        out_type=jax.ShapeDtypeStruct((NC, N, H), jnp.float32),
        scratch_types=(
            [pltpu.VMEM((CHUNK,), jnp.int32),
             pltpu.VMEM((CHUNK,), jnp.int32),
             pltpu.VMEM((CHUNK, 16), jnp.float32),
             pltpu.VMEM((CHUNK, H), jnp.float32),
             pltpu.VMEM((CHUNK, H), jnp.float32),
             pltpu.VMEM((CHUNK, H), jnp.float32)] * 2
            + [pltpu.VMEM((4, H), jnp.float32),
               pltpu.VMEM_SHARED((N, H), jnp.float32),
               pltpu.SemaphoreType.DMA,
               pltpu.SemaphoreType.DMA]
        ),
    )(_sc_edge_body)
    return fn(k, q, v, src, dst, ea16, wetb, zrows)


def kernel(x, edge_index, edge_attr, batch, clinical, params):
    src = edge_index[0].astype(jnp.int32)
    dst = edge_index[1].astype(jnp.int32)
    ea16 = jnp.pad(edge_attr, ((0, 0), (0, 12)))
    zrows = jnp.zeros((SRP, H), jnp.float32)

    a = x
    st = None
    gb = None
    for i in range(3):
        p = params[f"conv{i + 1}"]
        w4t = jnp.concatenate(
            [-p["Wk"], -p["Wq"], p["Wv"], p["Ws"]], axis=0).T
        b4 = jnp.concatenate(
            [-(p["bk"] + p["be"]), -p["bq"], p["bv"], p["b"]],
            axis=0)[None, :]
        if i == 0:
            k, q, v, s = _mm_call(a, w4t, b4)
        else:
            k, q, v, s = _norm_mm_call(a, st, gb, w4t, b4)
        wetb = -p["We"].T
        parts = _sc_edge_call(k, q, v, src, dst, ea16, wetb, zrows)
        a, st = _post_call(parts, s)
        gb = jnp.stack([p["gamma"], p["beta"]], axis=0)

    wc = params["Wc"]
    wht = wc[:, :H].T
    wct = wc[:, H:].T
    bc = params["bc"][None, :]
    out = _final_call(a, st, gb, batch[:, None].astype(jnp.int32),
                      clinical, wht, wct, bc)
    return jnp.squeeze(out)


# speedup vs baseline: 1.3625x; 1.3625x over previous
import functools

import jax
import jax.numpy as jnp
from jax import lax
from jax.experimental import pallas as pl
from jax.experimental.pallas import tpu as pltpu
from jax.experimental.pallas import tpu_sc as plsc

N = 10000
E = 320000
H = 128
B = 8
EPS = 1e-5
LOG2E = 1.4426950408889634

NB = 10
BN_ROWS = N // NB

NC = 2
NS = 16
NW = NC * NS
EPW = E // NW
CHUNK = 40
NCHUNK = EPW // CHUNK
SRP = 624
SREM = N - NS * SRP


def _mm_body(h_ref, w_ref, b_ref, k_ref, q_ref, v_ref, s_ref):
    y = jnp.dot(h_ref[...], w_ref[...], preferred_element_type=jnp.float32)
    y = y + b_ref[...]
    k_ref[...] = y[:, 0 * H:1 * H]
    q_ref[...] = y[:, 1 * H:2 * H]
    v_ref[...] = y[:, 2 * H:3 * H]
    s_ref[...] = y[:, 3 * H:4 * H]


def _norm_mm_body(a_ref, st_ref, gb_ref, w_ref, b_ref,
                  k_ref, q_ref, v_ref, s_ref):
    m = st_ref[0] * (1.0 / N)
    var = st_ref[1] * (1.0 / N) - m * m
    inv = lax.rsqrt(var + EPS)
    h = (a_ref[...] - m) * (inv * gb_ref[0]) + gb_ref[1]
    y = jnp.dot(h, w_ref[...], preferred_element_type=jnp.float32)
    y = y + b_ref[...]
    k_ref[...] = y[:, 0 * H:1 * H]
    q_ref[...] = y[:, 1 * H:2 * H]
    v_ref[...] = y[:, 2 * H:3 * H]
    s_ref[...] = y[:, 3 * H:4 * H]


def _post_body(p_ref, s_ref, a_ref, st_ref):
    i = pl.program_id(0)
    t = p_ref[0] + p_ref[1] + s_ref[...]
    a = jnp.where(t >= 0, t, 0.01 * t)
    a_ref[...] = a
    ssum = jnp.sum(a, axis=0, keepdims=True)
    ssq = jnp.sum(a * a, axis=0, keepdims=True)
    st = jnp.concatenate([ssum, ssq], axis=0)

    @pl.when(i == 0)
    def _():
        st_ref[...] = st

    @pl.when(i != 0)
    def _():
        st_ref[...] += st


def _final_body(a_ref, st_ref, gb_ref, batch_ref, clin_ref,
                wh_ref, wc_ref, bc_ref, o_ref):
    m = st_ref[0] * (1.0 / N)
    var = st_ref[1] * (1.0 / N) - m * m
    inv = lax.rsqrt(var + EPS)
    hn = (a_ref[...] - m) * (inv * gb_ref[0]) + gb_ref[1]
    oh = (batch_ref[...] == lax.broadcasted_iota(jnp.int32, (N, B), 1))
    oh = oh.astype(jnp.float32)
    psum = lax.dot_general(oh, hn, (((0,), (0,)), ((), ())),
                           preferred_element_type=jnp.float32)
    cnt = jnp.sum(oh, axis=0)[:, None]
    pooled = psum / jnp.clip(cnt, 1.0, None)
    out = jnp.dot(pooled, wh_ref[...], preferred_element_type=jnp.float32)
    out = out + jnp.dot(clin_ref[...], wc_ref[...],
                        preferred_element_type=jnp.float32)
    o_ref[...] = out + bc_ref[...]


def _mm_call(h, w4t, b4):
    return pl.pallas_call(
        _mm_body,
        grid=(NB,),
        in_specs=[
            pl.BlockSpec((BN_ROWS, H), lambda i: (i, 0)),
            pl.BlockSpec((H, 4 * H), lambda i: (0, 0)),
            pl.BlockSpec((1, 4 * H), lambda i: (0, 0)),
        ],
        out_specs=[pl.BlockSpec((BN_ROWS, H), lambda i: (i, 0))] * 4,
        out_shape=[jax.ShapeDtypeStruct((N, H), jnp.float32)] * 4,
    )(h, w4t, b4)


def _norm_mm_call(a, st, gb, w4t, b4):
    return pl.pallas_call(
        _norm_mm_body,
        grid=(NB,),
        in_specs=[
            pl.BlockSpec((BN_ROWS, H), lambda i: (i, 0)),
            pl.BlockSpec((2, H), lambda i: (0, 0)),
            pl.BlockSpec((2, H), lambda i: (0, 0)),
            pl.BlockSpec((H, 4 * H), lambda i: (0, 0)),
            pl.BlockSpec((1, 4 * H), lambda i: (0, 0)),
        ],
        out_specs=[pl.BlockSpec((BN_ROWS, H), lambda i: (i, 0))] * 4,
        out_shape=[jax.ShapeDtypeStruct((N, H), jnp.float32)] * 4,
    )(a, st, gb, w4t, b4)


def _post_call(parts, s):
    return pl.pallas_call(
        _post_body,
        grid=(NB,),
        in_specs=[
            pl.BlockSpec((2, BN_ROWS, H), lambda i: (0, i, 0)),
            pl.BlockSpec((BN_ROWS, H), lambda i: (i, 0)),
        ],
        out_specs=[
            pl.BlockSpec((BN_ROWS, H), lambda i: (i, 0)),
            pl.BlockSpec((2, H), lambda i: (0, 0)),
        ],
        out_shape=[
            jax.ShapeDtypeStruct((N, H), jnp.float32),
            jax.ShapeDtypeStruct((2, H), jnp.float32),
        ],
    )(parts, s)


def _final_call(a, st, gb, batch2d, clinical, wht, wct, bc):
    return pl.pallas_call(
        _final_body,
        out_shape=jax.ShapeDtypeStruct((B, 2), jnp.float32),
    )(a, st, gb, batch2d, clinical, wht, wct, bc)


def _sc_edge_body(k_hbm, q_hbm, v_hbm, src_hbm, dst_hbm, ea_hbm, we_hbm,
                  z_hbm, out_hbm,
                  sidx0, didx0, eav0, kb0, qb0, vb0,
                  sidx1, didx1, eav1, kb1, qb1, vb1,
                  wev, agg, sem0, sem1):
    cid = lax.axis_index("c")
    sid = lax.axis_index("s")
    wid = sid * NC + cid
    sets = ((sidx0, didx0, eav0, kb0, qb0, vb0, sem0),
            (sidx1, didx1, eav1, kb1, qb1, vb1, sem1))

    pltpu.sync_copy(we_hbm, wev)

    zbase = pl.multiple_of(sid * SRP, 8)
    pltpu.sync_copy(z_hbm.at[pl.ds(0, SRP)], agg.at[pl.ds(zbase, SRP)])

    @pl.when(sid == NS - 1)
    def _():
        pltpu.sync_copy(z_hbm.at[pl.ds(0, SREM)],
                        agg.at[pl.ds(NS * SRP, SREM)])

    plsc.subcore_barrier()

    we_rows = [[wev[c, pl.ds(s * 16, 16)] for s in range(8)]
               for c in range(4)]

    ebase = wid * EPW

    def load_idx(g, st):
        base = ebase + g * CHUNK
        pltpu.sync_copy(src_hbm.at[pl.ds(base, CHUNK)], st[0])
        pltpu.sync_copy(dst_hbm.at[pl.ds(base, CHUNK)], st[1])
        pltpu.sync_copy(ea_hbm.at[pl.ds(base, CHUNK)], st[2])

    def fire_gathers(st):
        pltpu.async_copy(k_hbm.at[st[1]], st[3], st[6])
        pltpu.async_copy(q_hbm.at[st[0]], st[4], st[6])
        pltpu.async_copy(v_hbm.at[st[0]], st[5], st[6])

    def wait_gathers(st):
        pltpu.make_async_copy(k_hbm.at[st[1]], st[3], st[6]).wait()
        pltpu.make_async_copy(q_hbm.at[st[0]], st[4], st[6]).wait()
        pltpu.make_async_copy(v_hbm.at[st[0]], st[5], st[6]).wait()

    def process(st):
        eav, kb, qb, vb = st[2], st[3], st[4], st[5]

        @plsc.parallel_loop(0, CHUNK, step=1, unroll=1)
        def edge_body(j):
            ear = eav[j, pl.ds(0, 16)]
            ea0 = ear[0]
            ea1 = ear[1]
            ea2 = ear[2]
            ea3 = ear[3]
            for s in range(8):
                ds = pl.ds(s * 16, 16)
                t01 = ea0 * we_rows[0][s] + ea1 * we_rows[1][s]
                t23 = ea2 * we_rows[2][s] + ea3 * we_rows[3][s]
                kq = kb[j, ds] + qb[j, ds]
                zn = (t01 + t23) + kq
                gate = 1.0 / (1.0 + jnp.exp(zn))
                vb[j, ds] = gate * vb[j, ds]

        pltpu.sync_copy(vb, agg.at[st[1]], add=True)

    load_idx(0, sets[0])
    fire_gathers(sets[0])

    def pair_body(g2, carry):
        for b in range(2):
            gi = g2 * 2 + b
            nxt = sets[1 - b]

            @pl.when(gi + 1 < NCHUNK)
            def _():
                load_idx(gi + 1, nxt)
                fire_gathers(nxt)

            wait_gathers(sets[b])
            process(sets[b])
        return carry

    lax.fori_loop(0, NCHUNK // 2, pair_body, 0)
    plsc.subcore_barrier()
    obase = pl.multiple_of(sid * SRP, 8)
    pltpu.sync_copy(agg.at[pl.ds(obase, SRP)],
                    out_hbm.at[cid, pl.ds(obase, SRP)])

    @pl.when(sid == NS - 1)
    def _():
        pltpu.sync_copy(agg.at[pl.ds(NS * SRP, SREM)],
                        out_hbm.at[cid, pl.ds(NS * SRP, SREM)])


def _sc_edge_call(k, q, v, src, dst, ea16, wetb, zrows):
    mesh = plsc.VectorSubcoreMesh(core_axis_name="c", subcore_axis_name="s")
    fn = functools.partial(
        pl.kernel,
        mesh=mesh,
        out_type=jax.ShapeDtypeStruct((NC, N, H), jnp.float32),
        scratch_types=(
            [pltpu.VMEM((CHUNK,), jnp.int32),
             pltpu.VMEM((CHUNK,), jnp.int32),
             pltpu.VMEM((CHUNK, 16), jnp.float32),
             pltpu.VMEM((CHUNK, H), jnp.float32),
             pltpu.VMEM((CHUNK, H), jnp.float32),
             pltpu.VMEM((CHUNK, H), jnp.float32)] * 2
            + [pltpu.VMEM((4, H), jnp.float32),
               pltpu.VMEM_SHARED((N, H), jnp.float32),
               pltpu.SemaphoreType.DMA,
               pltpu.SemaphoreType.DMA]
        ),
    )(_sc_edge_body)
    return fn(k, q, v, src, dst, ea16, wetb, zrows)


def kernel(x, edge_index, edge_attr, batch, clinical, params):
    src = edge_index[0].astype(jnp.int32)
    dst = edge_index[1].astype(jnp.int32)
    ea16 = jnp.pad(edge_attr, ((0, 0), (0, 12)))
    zrows = jnp.zeros((SRP, H), jnp.float32)

    a = x
    st = None
    gb = None
    for i in range(3):
        p = params[f"conv{i + 1}"]
        w4t = jnp.concatenate(
            [-p["Wk"], -p["Wq"], p["Wv"], p["Ws"]], axis=0).T
        b4 = jnp.concatenate(
            [-(p["bk"] + p["be"]), -p["bq"], p["bv"], p["b"]],
            axis=0)[None, :]
        if i == 0:
            k, q, v, s = _mm_call(a, w4t, b4)
        else:
            k, q, v, s = _norm_mm_call(a, st, gb, w4t, b4)
        wetb = -p["We"].T
        parts = _sc_edge_call(k, q, v, src, dst, ea16, wetb, zrows)
        a, st = _post_call(parts, s)
        gb = jnp.stack([p["gamma"], p["beta"]], axis=0)

    wc = params["Wc"]
    wht = wc[:, :H].T
    wct = wc[:, H:].T
    bc = params["bc"][None, :]
    out = _final_call(a, st, gb, batch[:, None].astype(jnp.int32),
                      clinical, wht, wct, bc)
    return jnp.squeeze(out)


# speedup vs baseline: 1.4050x; 1.0312x over previous
import functools

import jax
import jax.numpy as jnp
from jax import lax
from jax.experimental import pallas as pl
from jax.experimental.pallas import tpu as pltpu
from jax.experimental.pallas import tpu_sc as plsc

N = 10000
E = 320000
H = 128
B = 8
EPS = 1e-5
LOG2E = 1.4426950408889634

NB = 10
BN_ROWS = N // NB

NC = 2
NS = 16
NW = NC * NS
EPW = E // NW
CHUNK = 40
NCHUNK = EPW // CHUNK
SRP = 624
SREM = N - NS * SRP


def _mm_body(h_ref, w_ref, b_ref, k_ref, q_ref, v_ref, s_ref):
    y = jnp.dot(h_ref[...], w_ref[...], preferred_element_type=jnp.float32)
    y = y + b_ref[...]
    k_ref[...] = y[:, 0 * H:1 * H]
    q_ref[...] = y[:, 1 * H:2 * H]
    v_ref[...] = y[:, 2 * H:3 * H]
    s_ref[...] = y[:, 3 * H:4 * H]


def _norm_mm_body(a_ref, st_ref, gb_ref, w_ref, b_ref,
                  k_ref, q_ref, v_ref, s_ref):
    m = st_ref[0] * (1.0 / N)
    var = st_ref[1] * (1.0 / N) - m * m
    inv = lax.rsqrt(var + EPS)
    h = (a_ref[...] - m) * (inv * gb_ref[0]) + gb_ref[1]
    y = jnp.dot(h, w_ref[...], preferred_element_type=jnp.float32)
    y = y + b_ref[...]
    k_ref[...] = y[:, 0 * H:1 * H]
    q_ref[...] = y[:, 1 * H:2 * H]
    v_ref[...] = y[:, 2 * H:3 * H]
    s_ref[...] = y[:, 3 * H:4 * H]


def _post_body(p_ref, s_ref, a_ref, st_ref):
    i = pl.program_id(0)
    t = p_ref[0] + p_ref[1] + s_ref[...]
    a = jnp.where(t >= 0, t, 0.01 * t)
    a_ref[...] = a
    ssum = jnp.sum(a, axis=0, keepdims=True)
    ssq = jnp.sum(a * a, axis=0, keepdims=True)
    st = jnp.concatenate([ssum, ssq], axis=0)

    @pl.when(i == 0)
    def _():
        st_ref[...] = st

    @pl.when(i != 0)
    def _():
        st_ref[...] += st


def _final_body(a_ref, st_ref, gb_ref, batch_ref, clin_ref,
                wh_ref, wc_ref, bc_ref, o_ref):
    m = st_ref[0] * (1.0 / N)
    var = st_ref[1] * (1.0 / N) - m * m
    inv = lax.rsqrt(var + EPS)
    hn = (a_ref[...] - m) * (inv * gb_ref[0]) + gb_ref[1]
    oh = (batch_ref[...] == lax.broadcasted_iota(jnp.int32, (N, B), 1))
    oh = oh.astype(jnp.float32)
    psum = lax.dot_general(oh, hn, (((0,), (0,)), ((), ())),
                           preferred_element_type=jnp.float32)
    cnt = jnp.sum(oh, axis=0)[:, None]
    pooled = psum / jnp.clip(cnt, 1.0, None)
    out = jnp.dot(pooled, wh_ref[...], preferred_element_type=jnp.float32)
    out = out + jnp.dot(clin_ref[...], wc_ref[...],
                        preferred_element_type=jnp.float32)
    o_ref[...] = out + bc_ref[...]


def _mm_call(h, w4t, b4):
    return pl.pallas_call(
        _mm_body,
        grid=(NB,),
        in_specs=[
            pl.BlockSpec((BN_ROWS, H), lambda i: (i, 0)),
            pl.BlockSpec((H, 4 * H), lambda i: (0, 0)),
            pl.BlockSpec((1, 4 * H), lambda i: (0, 0)),
        ],
        out_specs=[pl.BlockSpec((BN_ROWS, H), lambda i: (i, 0))] * 4,
        out_shape=[jax.ShapeDtypeStruct((N, H), jnp.float32)] * 4,
    )(h, w4t, b4)


def _norm_mm_call(a, st, gb, w4t, b4):
    return pl.pallas_call(
        _norm_mm_body,
        grid=(NB,),
        in_specs=[
            pl.BlockSpec((BN_ROWS, H), lambda i: (i, 0)),
            pl.BlockSpec((2, H), lambda i: (0, 0)),
            pl.BlockSpec((2, H), lambda i: (0, 0)),
            pl.BlockSpec((H, 4 * H), lambda i: (0, 0)),
            pl.BlockSpec((1, 4 * H), lambda i: (0, 0)),
        ],
        out_specs=[pl.BlockSpec((BN_ROWS, H), lambda i: (i, 0))] * 4,
        out_shape=[jax.ShapeDtypeStruct((N, H), jnp.float32)] * 4,
    )(a, st, gb, w4t, b4)


def _post_call(parts, s):
    return pl.pallas_call(
        _post_body,
        grid=(NB,),
        in_specs=[
            pl.BlockSpec((2, BN_ROWS, H), lambda i: (0, i, 0)),
            pl.BlockSpec((BN_ROWS, H), lambda i: (i, 0)),
        ],
        out_specs=[
            pl.BlockSpec((BN_ROWS, H), lambda i: (i, 0)),
            pl.BlockSpec((2, H), lambda i: (0, 0)),
        ],
        out_shape=[
            jax.ShapeDtypeStruct((N, H), jnp.float32),
            jax.ShapeDtypeStruct((2, H), jnp.float32),
        ],
    )(parts, s)


def _final_call(a, st, gb, batch2d, clinical, wht, wct, bc):
    return pl.pallas_call(
        _final_body,
        out_shape=jax.ShapeDtypeStruct((B, 2), jnp.float32),
    )(a, st, gb, batch2d, clinical, wht, wct, bc)


def _sc_edge_body(k_hbm, q_hbm, v_hbm, src_hbm, dst_hbm, ea_hbm, we_hbm,
                  z_hbm, out_hbm,
                  sidx0, didx0, eav0, kq0, vb0, semk0, semq0, semv0,
                  sidx1, didx1, eav1, kq1, vb1, semk1, semq1, semv1,
                  wev, agg):
    cid = lax.axis_index("c")
    sid = lax.axis_index("s")
    wid = sid * NC + cid
    sets = ((sidx0, didx0, eav0, kq0, vb0, semk0, semq0, semv0),
            (sidx1, didx1, eav1, kq1, vb1, semk1, semq1, semv1))

    pltpu.sync_copy(we_hbm, wev)

    zbase = pl.multiple_of(sid * SRP, 8)
    pltpu.sync_copy(z_hbm.at[pl.ds(0, SRP)], agg.at[pl.ds(zbase, SRP)])

    @pl.when(sid == NS - 1)
    def _():
        pltpu.sync_copy(z_hbm.at[pl.ds(0, SREM)],
                        agg.at[pl.ds(NS * SRP, SREM)])

    plsc.subcore_barrier()

    we_rows = [[wev[c, pl.ds(s * 16, 16)] for s in range(8)]
               for c in range(4)]

    ebase = wid * EPW

    def load_idx(g, st):
        base = ebase + g * CHUNK
        pltpu.sync_copy(src_hbm.at[pl.ds(base, CHUNK)], st[0])
        pltpu.sync_copy(dst_hbm.at[pl.ds(base, CHUNK)], st[1])
        pltpu.sync_copy(ea_hbm.at[pl.ds(base, CHUNK)], st[2])

    def fire_kv(st):
        pltpu.async_copy(k_hbm.at[st[1]], st[3], st[5])
        pltpu.async_copy(v_hbm.at[st[0]], st[4], st[7])

    def fire_qadd(st):
        pltpu.async_copy(q_hbm.at[st[0]], st[3], st[6], add=True)

    def wait_k(st):
        pltpu.make_async_copy(k_hbm.at[st[1]], st[3], st[5]).wait()

    def wait_qv(st):
        pltpu.make_async_copy(q_hbm.at[st[0]], st[3], st[6]).wait()
        pltpu.make_async_copy(v_hbm.at[st[0]], st[4], st[7]).wait()

    def process(st):
        eav, kqb, vb = st[2], st[3], st[4]

        def edge_body(j, ecarry):
            ear = eav[j, pl.ds(0, 16)]
            ea0 = ear[0]
            ea1 = ear[1]
            ea2 = ear[2]
            ea3 = ear[3]
            for s in range(8):
                ds = pl.ds(s * 16, 16)
                t01 = ea0 * we_rows[0][s] + ea1 * we_rows[1][s]
                t23 = ea2 * we_rows[2][s] + ea3 * we_rows[3][s]
                zn = (t01 + t23) + kqb[j, ds]
                gate = 1.0 / (1.0 + jnp.exp(zn))
                vb[j, ds] = gate * vb[j, ds]
            return ecarry

        lax.fori_loop(0, CHUNK, edge_body, 0)

    load_idx(0, sets[0])
    fire_kv(sets[0])
    wait_k(sets[0])
    fire_qadd(sets[0])

    def pair_body(g2, carry):
        for b in range(2):
            gi = g2 * 2 + b
            cur = sets[b]
            nxt = sets[1 - b]

            @pl.when(gi + 1 < NCHUNK)
            def _():
                load_idx(gi + 1, nxt)
                fire_kv(nxt)

            wait_qv(cur)
            process(cur)

            @pl.when(gi + 1 < NCHUNK)
            def _():
                wait_k(nxt)
                fire_qadd(nxt)

            pltpu.sync_copy(cur[4], agg.at[cur[1]], add=True)
        return carry

    lax.fori_loop(0, NCHUNK // 2, pair_body, 0)
    plsc.subcore_barrier()
    obase = pl.multiple_of(sid * SRP, 8)
    pltpu.sync_copy(agg.at[pl.ds(obase, SRP)],
                    out_hbm.at[cid, pl.ds(obase, SRP)])

    @pl.when(sid == NS - 1)
    def _():
        pltpu.sync_copy(agg.at[pl.ds(NS * SRP, SREM)],
                        out_hbm.at[cid, pl.ds(NS * SRP, SREM)])


def _sc_edge_call(k, q, v, src, dst, ea16, wetb, zrows):
    mesh = plsc.VectorSubcoreMesh(core_axis_name="c", subcore_axis_name="s")
    fn = functools.partial(
        pl.kernel,
        mesh=mesh,
        out_type=jax.ShapeDtypeStruct((NC, N, H), jnp.float32),
        scratch_types=(
            [pltpu.VMEM((CHUNK,), jnp.int32),
             pltpu.VMEM((CHUNK,), jnp.int32),
             pltpu.VMEM((CHUNK, 16), jnp.float32),
             pltpu.VMEM((CHUNK, H), jnp.float32),
             pltpu.VMEM((CHUNK, H), jnp.float32),
             pltpu.SemaphoreType.DMA,
             pltpu.SemaphoreType.DMA,
             pltpu.SemaphoreType.DMA] * 2
            + [pltpu.VMEM((4, H), jnp.float32),
               pltpu.VMEM_SHARED((N, H), jnp.float32)]
        ),
    )(_sc_edge_body)
    return fn(k, q, v, src, dst, ea16, wetb, zrows)


def kernel(x, edge_index, edge_attr, batch, clinical, params):
    src = edge_index[0].astype(jnp.int32)
    dst = edge_index[1].astype(jnp.int32)
    ea16 = jnp.pad(edge_attr, ((0, 0), (0, 12)))
    zrows = jnp.zeros((SRP, H), jnp.float32)

    a = x
    st = None
    gb = None
    for i in range(3):
        p = params[f"conv{i + 1}"]
        w4t = jnp.concatenate(
            [-p["Wk"], -p["Wq"], p["Wv"], p["Ws"]], axis=0).T
        b4 = jnp.concatenate(
            [-(p["bk"] + p["be"]), -p["bq"], p["bv"], p["b"]],
            axis=0)[None, :]
        if i == 0:
            k, q, v, s = _mm_call(a, w4t, b4)
        else:
            k, q, v, s = _norm_mm_call(a, st, gb, w4t, b4)
        wetb = -p["We"].T
        parts = _sc_edge_call(k, q, v, src, dst, ea16, wetb, zrows)
        a, st = _post_call(parts, s)
        gb = jnp.stack([p["gamma"], p["beta"]], axis=0)

    wc = params["Wc"]
    wht = wc[:, :H].T
    wct = wc[:, H:].T
    bc = params["bc"][None, :]
    out = _final_call(a, st, gb, batch[:, None].astype(jnp.int32),
                      clinical, wht, wct, bc)
    return jnp.squeeze(out)
